# baseline (speedup 1.0000x reference)
import functools

import jax
import jax.numpy as jnp
from jax import lax
from jax.experimental import pallas as pl
from jax.experimental.pallas import tpu as pltpu
from jax.experimental.pallas import tpu_sc as plsc


_NC = 2
_NS = 16
_NW = _NC * _NS
_CH = 400
_NBUF = 4


def kernel(x):
    n, d = x.shape
    nch = n // _CH
    mesh = plsc.VectorSubcoreMesh(core_axis_name="c", subcore_axis_name="s")

    @functools.partial(
        pl.kernel,
        out_type=jax.ShapeDtypeStruct((n, d), x.dtype),
        mesh=mesh,
        scratch_types=[
            pltpu.VMEM((_NBUF, _CH, d), x.dtype),
            pltpu.SemaphoreType.DMA((_NBUF,)),
            pltpu.SemaphoreType.DMA((_NBUF,)),
        ],
        compiler_params=pltpu.CompilerParams(use_tc_tiling_on_sc=False),
    )
    def _copy(x_hbm, o_hbm, bufs, in_sems, out_sems):
        wid = lax.axis_index("s") * _NC + lax.axis_index("c")
        n_my = (nch - wid + _NW - 1) // _NW

        def in_start(b, k):
            row = (wid + k * _NW) * _CH
            pltpu.make_async_copy(
                x_hbm.at[pl.ds(row, _CH), :], bufs.at[b], in_sems.at[b]
            ).start()

        for b in range(_NBUF):
            @pl.when(b < n_my)
            def _():
                in_start(b, b)

        def step(k, carry):
            b = lax.rem(k, _NBUF)
            row = (wid + k * _NW) * _CH
            pltpu.make_async_copy(
                x_hbm.at[pl.ds(row, _CH), :], bufs.at[b], in_sems.at[b]
            ).wait()

            @pl.when(jnp.logical_and(wid == 0, k == 0))
            def _():
                ones = jnp.ones((16,), x.dtype)
                for r in range(2):
                    for j in range(d // 16):
                        bufs[0, r, pl.ds(16 * j, 16)] = ones

            out_cp = pltpu.make_async_copy(
                bufs.at[b], o_hbm.at[pl.ds(row, _CH), :], out_sems.at[b]
            )
            out_cp.start()
            out_cp.wait()

            @pl.when(k + _NBUF < n_my)
            def _():
                in_start(b, k + _NBUF)

            return carry

        lax.fori_loop(0, n_my, step, 0)

    return _copy(x)
